# Initial kernel scaffold; baseline (speedup 1.0000x reference)
import jax
import jax.numpy as jnp
from jax.experimental import pallas as pl


def kernel(x, edge_index, W1, b1, W2, b2):
    raise NotImplementedError("write your pallas kernel here")


# speedup vs baseline: 7.1773x; 7.1773x over previous
import dataclasses
import functools

import jax
import jax.numpy as jnp
from jax import lax
from jax.experimental import pallas as pl
from jax.experimental.pallas import tpu as pltpu
from jax.experimental.pallas import tpu_sc as plsc

N = 10000
E = 320000
D_IN = 128
D_HID = 256
D_OUT = 128

NPAD = 10240
E_PAD = 327680
NC = 2
NS = 16
BLK = 128
NB1 = 160
NB2 = 80
IDXCH = 16
ROWS_PER_TILE = NPAD // NS

R = 400
G = N // R

_mesh = plsc.VectorSubcoreMesh(core_axis_name="c", subcore_axis_name="s")
_f32 = jnp.float32


_sc_cp = pltpu.CompilerParams()
if "needs_layout_passes" in pltpu.CompilerParams.__dataclass_fields__:
    _sc_cp = dataclasses.replace(_sc_cp, needs_layout_passes=False)


@functools.partial(
    pl.kernel,
    out_type=jax.ShapeDtypeStruct((NC * NS, NPAD), _f32),
    mesh=_mesh,
    compiler_params=_sc_cp,
    scratch_types=[
        pltpu.VMEM((NB2, BLK), jnp.int32),
        pltpu.VMEM((NPAD,), _f32),
    ],
)
def _deg_kernel(dst_hbm, zeros_hbm, out_hbm, dst_v, hist_v):
    c = lax.axis_index("c")
    s = lax.axis_index("s")
    wid = c * NS + s

    pltpu.sync_copy(dst_hbm.at[wid], dst_v)
    pltpu.sync_copy(zeros_hbm, hist_v)
    ones = jnp.ones((16,), _f32)

    @pl.loop(0, NB2)
    def _(j):
        for k in range(BLK // 16):
            idx = dst_v[j, pl.ds(k * 16, 16)]
            plsc.addupdate_scatter(hist_v, [idx], ones)

    pltpu.sync_copy(hist_v, out_hbm.at[wid])


def _make_acc_kernel(nblk, col_split):
    @functools.partial(
        pl.kernel,
        out_type=jax.ShapeDtypeStruct((NC, NPAD, 128), _f32),
        mesh=_mesh,
        scratch_types=[
            pltpu.VMEM((IDXCH, BLK), jnp.int32),
            pltpu.VMEM((IDXCH, BLK), jnp.int32),
            pltpu.VMEM((BLK, 128), _f32),
            pltpu.VMEM_SHARED((NPAD, 128), _f32),
            pltpu.SemaphoreType.DMA,
        ],
    )
    def _acc_kernel(table_hbm, src_hbm, dst_hbm, zeros_hbm, out_hbm,
                    src_v, dst_v, rows_v, acc_sh, sem):
        c = lax.axis_index("c")
        s = lax.axis_index("s")
        idx_row = s if col_split else c * NS + s

        pltpu.sync_copy(zeros_hbm,
                        acc_sh.at[pl.ds(s * ROWS_PER_TILE, ROWS_PER_TILE)])
        plsc.subcore_barrier()

        table = table_hbm.at[c] if col_split else table_hbm

        @pl.loop(0, nblk // IDXCH)
        def _(jc):
            csl = pl.ds(jc * IDXCH, IDXCH)
            pltpu.sync_copy(src_hbm.at[idx_row, csl], src_v)
            pltpu.sync_copy(dst_hbm.at[idx_row, csl], dst_v)

            @pl.loop(0, IDXCH)
            def _(j):
                pltpu.async_copy(table.at[src_v.at[j]], rows_v, sem).wait()
                pltpu.sync_copy(rows_v, acc_sh.at[dst_v.at[j]], add=True)

        plsc.subcore_barrier()
        sl = pl.ds(s * ROWS_PER_TILE, ROWS_PER_TILE)
        pltpu.sync_copy(acc_sh.at[sl], out_hbm.at[c, sl])

    return _acc_kernel


_acc_colsplit = _make_acc_kernel(NB1, col_split=True)
_acc_edgesplit = _make_acc_kernel(NB2, col_split=False)


def _dinv_body(deg_ref, o_ref):
    deg = jnp.sum(deg_ref[...], axis=0) + 1.0
    o_ref[...] = lax.rsqrt(deg)[:, None]


def _dinv(degp):
    return pl.pallas_call(
        _dinv_body,
        out_shape=jax.ShapeDtypeStruct((NPAD, 1), _f32),
    )(degp)


def _mm1_body(x_ref, w_ref, o_ref):
    o_ref[...] = jnp.dot(x_ref[...], w_ref[...],
                         preferred_element_type=_f32)[None]


def _mm1(x, W1):
    return pl.pallas_call(
        _mm1_body,
        grid=(G, NC),
        in_specs=[
            pl.BlockSpec((R, D_IN), lambda i, j: (i, 0)),
            pl.BlockSpec((D_IN, 128), lambda i, j: (0, j)),
        ],
        out_specs=pl.BlockSpec((1, R, 128), lambda i, j: (j, i, 0)),
        out_shape=jax.ShapeDtypeStruct((NC, N, 128), _f32),
    )(x, W1)


def _scale_body(m_ref, d_ref, o_ref):
    o_ref[...] = m_ref[...] * d_ref[...][None]


def _scale(m1, dinv):
    return pl.pallas_call(
        _scale_body,
        grid=(G, NC),
        in_specs=[
            pl.BlockSpec((1, R, 128), lambda i, j: (j, i, 0)),
            pl.BlockSpec((R, 1), lambda i, j: (i, 0)),
        ],
        out_specs=pl.BlockSpec((1, R, 128), lambda i, j: (j, i, 0)),
        out_shape=jax.ShapeDtypeStruct((NC, N, 128), _f32),
    )(m1, dinv)


def _ep1_body(a_ref, h_ref, d_ref, b_ref, w_ref, h1o_ref, h2p_ref):
    dinv = d_ref[...]
    t = jnp.concatenate([a_ref[0] + h_ref[0], a_ref[1] + h_ref[1]], axis=-1)
    h1b = jnp.maximum(t * dinv + b_ref[...], 0.0)
    h1o_ref[...] = h1b
    h2p_ref[...] = jnp.dot(h1b, w_ref[...], preferred_element_type=_f32) * dinv


def _ep1(acc1, h1p, dinv, b1, W2):
    return pl.pallas_call(
        _ep1_body,
        grid=(G,),
        in_specs=[
            pl.BlockSpec((NC, R, 128), lambda i: (0, i, 0)),
            pl.BlockSpec((NC, R, 128), lambda i: (0, i, 0)),
            pl.BlockSpec((R, 1), lambda i: (i, 0)),
            pl.BlockSpec((1, D_HID), lambda i: (0, 0)),
            pl.BlockSpec((D_HID, D_OUT), lambda i: (0, 0)),
        ],
        out_specs=[
            pl.BlockSpec((R, D_HID), lambda i: (i, 0)),
            pl.BlockSpec((R, D_OUT), lambda i: (i, 0)),
        ],
        out_shape=[
            jax.ShapeDtypeStruct((N, D_HID), _f32),
            jax.ShapeDtypeStruct((N, D_OUT), _f32),
        ],
    )(acc1, h1p, dinv, b1, W2)


def _ep2_body(a_ref, h_ref, d_ref, b_ref, o_ref):
    t = a_ref[0] + a_ref[1] + h_ref[...]
    o_ref[...] = jnp.maximum(t * d_ref[...] + b_ref[...], 0.0)


def _ep2(acc2, h2p, dinv, b2):
    return pl.pallas_call(
        _ep2_body,
        grid=(G,),
        in_specs=[
            pl.BlockSpec((NC, R, 128), lambda i: (0, i, 0)),
            pl.BlockSpec((R, D_OUT), lambda i: (i, 0)),
            pl.BlockSpec((R, 1), lambda i: (i, 0)),
            pl.BlockSpec((1, D_OUT), lambda i: (0, 0)),
        ],
        out_specs=pl.BlockSpec((R, D_OUT), lambda i: (i, 0)),
        out_shape=jax.ShapeDtypeStruct((N, D_OUT), _f32),
    )(acc2, h2p, dinv, b2)


def kernel(x, edge_index, W1, b1, W2, b2):
    src = edge_index[0].astype(jnp.int32)
    dst = edge_index[1].astype(jnp.int32)

    pad = E_PAD - E
    src_p = jnp.concatenate([src, jnp.zeros((pad,), jnp.int32)])
    dst_p = jnp.concatenate(
        [dst, N + (jnp.arange(pad, dtype=jnp.int32) % (NPAD - N))])

    src1 = src_p.reshape(NS, NB1, BLK)
    dst1 = dst_p.reshape(NS, NB1, BLK)
    src2 = src_p.reshape(NC * NS, NB2, BLK)
    dst2 = dst_p.reshape(NC * NS, NB2, BLK)

    zeros1 = jnp.zeros((NPAD,), _f32)
    zeros128 = jnp.zeros((ROWS_PER_TILE, 128), _f32)

    degp = _deg_kernel(dst2, zeros1)
    m1 = _mm1(x, W1)
    dinv = _dinv(degp)
    h1p = _scale(m1, dinv)
    acc1 = _acc_colsplit(h1p, src1, dst1, zeros128)
    h1, h2p = _ep1(acc1, h1p, dinv, b1.reshape(1, D_HID), W2)
    acc2 = _acc_edgesplit(h2p, src2, dst2, zeros128)
    h2 = _ep2(acc2, h2p, dinv, b2.reshape(1, D_OUT))
    return (h1, h2)


# speedup vs baseline: 9.3361x; 1.3008x over previous
import dataclasses
import functools

import jax
import jax.numpy as jnp
from jax import lax
from jax.experimental import pallas as pl
from jax.experimental.pallas import tpu as pltpu
from jax.experimental.pallas import tpu_sc as plsc

N = 10000
E = 320000
D_IN = 128
D_HID = 256
D_OUT = 128

NPAD = 10240
E_PAD = 327680
NC = 2
NS = 16
BLK = 64
NB1 = 320
NB2 = 160
CH = 40
NBUF = 4
ROWS_PER_TILE = NPAD // NS

R = 400
G = N // R

_mesh = plsc.VectorSubcoreMesh(core_axis_name="c", subcore_axis_name="s")
_f32 = jnp.float32


_sc_cp = pltpu.CompilerParams()
if "needs_layout_passes" in pltpu.CompilerParams.__dataclass_fields__:
    _sc_cp = dataclasses.replace(_sc_cp, needs_layout_passes=False)


@functools.partial(
    pl.kernel,
    out_type=jax.ShapeDtypeStruct((NC * NS, NPAD), _f32),
    mesh=_mesh,
    compiler_params=_sc_cp,
    scratch_types=[
        pltpu.VMEM((NB2, BLK), jnp.int32),
        pltpu.VMEM((NPAD,), _f32),
    ],
)
def _deg_kernel(dst_hbm, zeros_hbm, out_hbm, dst_v, hist_v):
    c = lax.axis_index("c")
    s = lax.axis_index("s")
    wid = c * NS + s

    pltpu.sync_copy(dst_hbm.at[wid], dst_v)
    pltpu.sync_copy(zeros_hbm, hist_v)
    ones = jnp.ones((16,), _f32)

    @pl.loop(0, NB2)
    def _(j):
        for k in range(BLK // 16):
            idx = dst_v[j, pl.ds(k * 16, 16)]
            plsc.addupdate_scatter(hist_v, [idx], ones)

    pltpu.sync_copy(hist_v, out_hbm.at[wid])


def _make_acc_kernel(nblk, col_split):
    @functools.partial(
        pl.kernel,
        out_type=jax.ShapeDtypeStruct((NC, NPAD, 128), _f32),
        mesh=_mesh,
        scratch_types=[
            pltpu.VMEM((CH, BLK), jnp.int32),
            pltpu.VMEM((CH, BLK), jnp.int32),
            pltpu.VMEM((NBUF, BLK, 128), _f32),
            pltpu.VMEM_SHARED((NPAD, 128), _f32),
            pltpu.SemaphoreType.DMA((NBUF,)),
        ],
    )
    def _acc_kernel(table_hbm, src_hbm, dst_hbm, zeros_hbm, out_hbm,
                    src_v, dst_v, bufs_v, acc_sh, gsem):
        c = lax.axis_index("c")
        s = lax.axis_index("s")
        idx_row = s if col_split else c * NS + s

        pltpu.sync_copy(zeros_hbm,
                        acc_sh.at[pl.ds(s * ROWS_PER_TILE, ROWS_PER_TILE)])
        plsc.subcore_barrier()

        table = table_hbm.at[c] if col_split else table_hbm
        dummy_src = zeros_hbm.at[pl.ds(0, BLK)]

        @pl.loop(0, nblk // CH)
        def _(t):
            csl = pl.ds(t * CH, CH)
            pltpu.sync_copy(src_hbm.at[idx_row, csl], src_v)
            pltpu.sync_copy(dst_hbm.at[idx_row, csl], dst_v)
            for b in range(NBUF):
                pltpu.async_copy(table.at[src_v.at[b]], bufs_v.at[b],
                                 gsem.at[b])

            @pl.loop(0, CH // NBUF)
            def _(q):
                for b in range(NBUF):
                    jj = q * NBUF + b
                    pltpu.make_async_copy(dummy_src, bufs_v.at[b],
                                          gsem.at[b]).wait()
                    pltpu.sync_copy(bufs_v.at[b], acc_sh.at[dst_v.at[jj]],
                                    add=True)

                    @pl.when(jj < CH - NBUF)
                    def _():
                        pltpu.async_copy(table.at[src_v.at[jj + NBUF]],
                                         bufs_v.at[b], gsem.at[b])

        plsc.subcore_barrier()
        sl = pl.ds(s * ROWS_PER_TILE, ROWS_PER_TILE)
        pltpu.sync_copy(acc_sh.at[sl], out_hbm.at[c, sl])

    return _acc_kernel


_acc_colsplit = _make_acc_kernel(NB1, col_split=True)
_acc_edgesplit = _make_acc_kernel(NB2, col_split=False)


def _dinv_body(deg_ref, o_ref):
    deg = jnp.sum(deg_ref[...], axis=0) + 1.0
    o_ref[...] = lax.rsqrt(deg)[:, None]


def _dinv(degp):
    return pl.pallas_call(
        _dinv_body,
        out_shape=jax.ShapeDtypeStruct((NPAD, 1), _f32),
    )(degp)


def _mm1_body(x_ref, w_ref, o_ref):
    o_ref[...] = jnp.dot(x_ref[...], w_ref[...],
                         preferred_element_type=_f32)[None]


def _mm1(x, W1):
    return pl.pallas_call(
        _mm1_body,
        grid=(G, NC),
        in_specs=[
            pl.BlockSpec((R, D_IN), lambda i, j: (i, 0)),
            pl.BlockSpec((D_IN, 128), lambda i, j: (0, j)),
        ],
        out_specs=pl.BlockSpec((1, R, 128), lambda i, j: (j, i, 0)),
        out_shape=jax.ShapeDtypeStruct((NC, N, 128), _f32),
    )(x, W1)


def _scale_body(m_ref, d_ref, o_ref):
    o_ref[...] = m_ref[...] * d_ref[...][None]


def _scale(m1, dinv):
    return pl.pallas_call(
        _scale_body,
        grid=(G, NC),
        in_specs=[
            pl.BlockSpec((1, R, 128), lambda i, j: (j, i, 0)),
            pl.BlockSpec((R, 1), lambda i, j: (i, 0)),
        ],
        out_specs=pl.BlockSpec((1, R, 128), lambda i, j: (j, i, 0)),
        out_shape=jax.ShapeDtypeStruct((NC, N, 128), _f32),
    )(m1, dinv)


def _ep1_body(a_ref, h_ref, d_ref, b_ref, w_ref, h1o_ref, h2p_ref):
    dinv = d_ref[...]
    t = jnp.concatenate([a_ref[0] + h_ref[0], a_ref[1] + h_ref[1]], axis=-1)
    h1b = jnp.maximum(t * dinv + b_ref[...], 0.0)
    h1o_ref[...] = h1b
    h2p_ref[...] = jnp.dot(h1b, w_ref[...], preferred_element_type=_f32) * dinv


def _ep1(acc1, h1p, dinv, b1, W2):
    return pl.pallas_call(
        _ep1_body,
        grid=(G,),
        in_specs=[
            pl.BlockSpec((NC, R, 128), lambda i: (0, i, 0)),
            pl.BlockSpec((NC, R, 128), lambda i: (0, i, 0)),
            pl.BlockSpec((R, 1), lambda i: (i, 0)),
            pl.BlockSpec((1, D_HID), lambda i: (0, 0)),
            pl.BlockSpec((D_HID, D_OUT), lambda i: (0, 0)),
        ],
        out_specs=[
            pl.BlockSpec((R, D_HID), lambda i: (i, 0)),
            pl.BlockSpec((R, D_OUT), lambda i: (i, 0)),
        ],
        out_shape=[
            jax.ShapeDtypeStruct((N, D_HID), _f32),
            jax.ShapeDtypeStruct((N, D_OUT), _f32),
        ],
    )(acc1, h1p, dinv, b1, W2)


def _ep2_body(a_ref, h_ref, d_ref, b_ref, o_ref):
    t = a_ref[0] + a_ref[1] + h_ref[...]
    o_ref[...] = jnp.maximum(t * d_ref[...] + b_ref[...], 0.0)


def _ep2(acc2, h2p, dinv, b2):
    return pl.pallas_call(
        _ep2_body,
        grid=(G,),
        in_specs=[
            pl.BlockSpec((NC, R, 128), lambda i: (0, i, 0)),
            pl.BlockSpec((R, D_OUT), lambda i: (i, 0)),
            pl.BlockSpec((R, 1), lambda i: (i, 0)),
            pl.BlockSpec((1, D_OUT), lambda i: (0, 0)),
        ],
        out_specs=pl.BlockSpec((R, D_OUT), lambda i: (i, 0)),
        out_shape=jax.ShapeDtypeStruct((N, D_OUT), _f32),
    )(acc2, h2p, dinv, b2)


def kernel(x, edge_index, W1, b1, W2, b2):
    src = edge_index[0].astype(jnp.int32)
    dst = edge_index[1].astype(jnp.int32)

    pad = E_PAD - E
    src_p = jnp.concatenate([src, jnp.zeros((pad,), jnp.int32)])
    dst_p = jnp.concatenate(
        [dst, N + (jnp.arange(pad, dtype=jnp.int32) % (NPAD - N))])

    src1 = src_p.reshape(NS, NB1, BLK)
    dst1 = dst_p.reshape(NS, NB1, BLK)
    src2 = src_p.reshape(NC * NS, NB2, BLK)
    dst2 = dst_p.reshape(NC * NS, NB2, BLK)

    zeros1 = jnp.zeros((NPAD,), _f32)
    zeros128 = jnp.zeros((ROWS_PER_TILE, 128), _f32)

    degp = _deg_kernel(dst2, zeros1)
    m1 = _mm1(x, W1)
    dinv = _dinv(degp)
    h1p = _scale(m1, dinv)
    acc1 = _acc_colsplit(h1p, src1, dst1, zeros128)
    h1, h2p = _ep1(acc1, h1p, dinv, b1.reshape(1, D_HID), W2)
    acc2 = _acc_edgesplit(h2p, src2, dst2, zeros128)
    h2 = _ep2(acc2, h2p, dinv, b2.reshape(1, D_OUT))
    return (h1, h2)
